# baseline (speedup 1.0000x reference)
import functools

import jax
import jax.numpy as jnp
from jax import lax
from jax.experimental import pallas as pl
from jax.experimental.pallas import tpu as pltpu
from jax.experimental.pallas import tpu_sc as plsc

N, D_IN, D_OUT, E, TOP_K = 4096, 1024, 1024, 8, 2
RT = 2048
LANES = 128
NEG = -1e30

NW = 32
NPAIR = TOP_K * N
CP = NPAIR // NW
TM = 256
P = NPAIR + E * TM
NT = P // TM
EOT_PAD = 48
TT5 = 2048


def _router_body(x_ref, wr_ref, br_ref, idx_ref, w_ref):
    lane = lax.broadcasted_iota(jnp.int32, (RT, LANES), 1)
    logits = jnp.dot(x_ref[...], wr_ref[...],
                     preferred_element_type=jnp.float32) + br_ref[...]
    logits = jnp.where(lane < E, logits, NEG)
    m1 = jnp.max(logits, axis=1, keepdims=True)
    i1 = jnp.min(jnp.where(logits == m1, lane, LANES), axis=1, keepdims=True)
    l2 = jnp.where(lane == i1, NEG, logits)
    m2 = jnp.max(l2, axis=1, keepdims=True)
    i2 = jnp.min(jnp.where(l2 == m2, lane, LANES), axis=1, keepdims=True)
    w0 = 1.0 / (1.0 + jnp.exp(m2 - m1))
    idx_ref[...] = jnp.concatenate([i1, i2], axis=1)
    w_ref[...] = jnp.concatenate([w0, 1.0 - w0], axis=1)


def _router(x, wr_pad, br_pad):
    return pl.pallas_call(
        _router_body,
        grid=(N // RT,),
        in_specs=[
            pl.BlockSpec((RT, D_IN), lambda t: (t, 0)),
            pl.BlockSpec((D_IN, LANES), lambda t: (0, 0)),
            pl.BlockSpec((1, LANES), lambda t: (0, 0)),
        ],
        out_specs=[
            pl.BlockSpec((RT, TOP_K), lambda t: (t, 0)),
            pl.BlockSpec((RT, TOP_K), lambda t: (t, 0)),
        ],
        out_shape=[
            jax.ShapeDtypeStruct((N, TOP_K), jnp.int32),
            jax.ShapeDtypeStruct((N, TOP_K), jnp.float32),
        ],
    )(x, wr_pad, br_pad)


def _dispatch_body(idx_hbm, x_hbm, pos_hbm, eot_hbm, xs_hbm,
                   idx_all, cur_v, off_v, ptot_v, pos_v, eot_v,
                   xbuf_a, xbuf_b, sem_ld, sem_st):
    wid = lax.axis_index("s") * 2 + lax.axis_index("c")
    lane16 = lax.broadcasted_iota(jnp.int32, (16,), 0)
    zeros = jnp.zeros((16,), jnp.int32)

    pltpu.sync_copy(idx_hbm, idx_all)

    def cbody(k, carry):
        tot, pre = carry
        v = idx_all[pl.ds(k * 16, 16)]
        ipre = jnp.where(k < wid * 16, 1, 0)
        for e in range(E):
            c = jnp.sum(jnp.where(v == e, 1, 0))
            sel = lane16 == e
            tot = tot + jnp.where(sel, c, 0)
            pre = pre + jnp.where(sel, c * ipre, 0)
        return tot, pre

    tot, pre = lax.fori_loop(0, NPAIR // 16, cbody, (zeros, zeros))

    ptot = ((tot + (TM - 1)) >> 8) << 8
    off_incl = plsc.cumsum(ptot)
    off_excl = off_incl - ptot
    cur = off_excl + pre

    for k in range(CP // 16):
        v = idx_all[pl.ds((wid * 16 + k) * 16, 16)]
        cur_v[...] = cur
        base = plsc.load_gather(cur_v, [v])
        rank = zeros
        for e in range(E):
            m = v == e
            c01 = jnp.where(m, 1, 0)
            pref = plsc.cumsum(c01)
            rank = jnp.where(m, pref - 1, rank)
            cnt = jnp.sum(c01)
            cur = cur + jnp.where(lane16 == e, cnt, 0)
        pos_v[k] = base + rank

    pltpu.sync_copy(pos_v, pos_hbm.at[pl.ds(wid * 16, 16)])

    @pl.when(wid == 0)
    def _eot():
        off_v[...] = off_excl
        ptot_v[...] = ptot
        for k in range(EOT_PAD // 16):
            tstart = (lane16 + 16 * k) << 8
            eid = zeros
            for e in range(E):
                fe = jnp.full((16,), e, jnp.int32)
                oe = plsc.load_gather(off_v, [fe])
                pe = plsc.load_gather(ptot_v, [fe])
                inb = (tstart >= oe) & (tstart < oe + pe)
                eid = jnp.where(inb, e, eid)
            eot_v[pl.ds(16 * k, 16)] = eid
        pltpu.sync_copy(eot_v, eot_hbm)

    tok0 = wid * CP
    bufs = [xbuf_a, xbuf_b]
    nst = CP // 16
    lds = [None] * nst
    sts = [None] * nst
    lds[0] = pltpu.async_copy(x_hbm.at[pl.ds(tok0, 16)], bufs[0], sem_ld)
    for k in range(nst):
        lds[k].wait()
        if k + 1 < nst:
            if k >= 1:
                sts[k - 1].wait()
            lds[k + 1] = pltpu.async_copy(
                x_hbm.at[pl.ds(tok0 + (k + 1) * 16, 16)],
                bufs[(k + 1) % 2], sem_ld)
        sts[k] = pltpu.async_copy(bufs[k % 2], xs_hbm.at[pos_v.at[k]], sem_st)
    sts[nst - 2].wait()
    sts[nst - 1].wait()


def _dispatch(idx_flat, x):
    mesh = plsc.VectorSubcoreMesh(core_axis_name="c", subcore_axis_name="s")
    f = functools.partial(
        pl.kernel, mesh=mesh,
        compiler_params=pltpu.CompilerParams(needs_layout_passes=False),
        out_type=[
            jax.ShapeDtypeStruct((NPAIR // 16, 16), jnp.int32),
            jax.ShapeDtypeStruct((EOT_PAD,), jnp.int32),
            jax.ShapeDtypeStruct((P, D_IN), jnp.float32),
        ],
        scratch_types=[
            pltpu.VMEM((NPAIR,), jnp.int32),
            pltpu.VMEM((16,), jnp.int32),
            pltpu.VMEM((16,), jnp.int32),
            pltpu.VMEM((16,), jnp.int32),
            pltpu.VMEM((16, 16), jnp.int32),
            pltpu.VMEM((EOT_PAD,), jnp.int32),
            pltpu.VMEM((16, D_IN), jnp.float32),
            pltpu.VMEM((16, D_IN), jnp.float32),
            pltpu.SemaphoreType.DMA,
            pltpu.SemaphoreType.DMA,
        ],
    )(_dispatch_body)
    return f(idx_flat, x)


def _ffn_body(eot_ref, xs_ref, w_ref, b_ref, ys_ref):
    z = jnp.dot(xs_ref[...], w_ref[0],
                preferred_element_type=jnp.float32) + b_ref[0]
    ys_ref[...] = 0.5 * z * (1.0 + lax.erf(z * 0.7071067811865476))


def _ffn(xs, eot, W, b):
    return pl.pallas_call(
        _ffn_body,
        grid_spec=pltpu.PrefetchScalarGridSpec(
            num_scalar_prefetch=1,
            grid=(NT,),
            in_specs=[
                pl.BlockSpec((TM, D_IN), lambda t, eot: (t, 0)),
                pl.BlockSpec((1, D_IN, D_OUT), lambda t, eot: (eot[t], 0, 0)),
                pl.BlockSpec((1, 1, D_OUT), lambda t, eot: (eot[t], 0, 0)),
            ],
            out_specs=pl.BlockSpec((TM, D_OUT), lambda t, eot: (t, 0)),
        ),
        out_shape=jax.ShapeDtypeStruct((P, D_OUT), jnp.float32),
        compiler_params=pltpu.CompilerParams(
            dimension_semantics=("arbitrary",),
        ),
    )(eot, xs, W, b.reshape(E, 1, D_OUT))


def _gather_body(ys_hbm, pos_hbm, y0_hbm, y1_hbm,
                 p0_v, p1_v, ybuf_a, ybuf_b, sem):
    wid = lax.axis_index("s") * 2 + lax.axis_index("c")
    rows0 = N // 16
    pltpu.sync_copy(pos_hbm.at[pl.ds(wid * 8, 8)], p0_v)
    pltpu.sync_copy(pos_hbm.at[pl.ds(rows0 + wid * 8, 8)], p1_v)
    bufs = [ybuf_a, ybuf_b]
    nst = 16
    gs = [None] * nst

    def idx_row(j):
        return p0_v.at[j] if j < 8 else p1_v.at[j - 8]

    gs[0] = pltpu.async_copy(ys_hbm.at[idx_row(0)], bufs[0], sem)
    for j in range(nst):
        gs[j].wait()
        if j + 1 < nst:
            gs[j + 1] = pltpu.async_copy(
                ys_hbm.at[idx_row(j + 1)], bufs[(j + 1) % 2], sem)
        if j < 8:
            dst = y0_hbm.at[pl.ds(wid * 128 + j * 16, 16)]
        else:
            dst = y1_hbm.at[pl.ds(wid * 128 + (j - 8) * 16, 16)]
        pltpu.sync_copy(bufs[j % 2], dst)


def _combine_gather(ys, pos):
    mesh = plsc.VectorSubcoreMesh(core_axis_name="c", subcore_axis_name="s")
    f = functools.partial(
        pl.kernel, mesh=mesh,
        compiler_params=pltpu.CompilerParams(needs_layout_passes=False),
        out_type=[
            jax.ShapeDtypeStruct((N, D_OUT), jnp.float32),
            jax.ShapeDtypeStruct((N, D_OUT), jnp.float32),
        ],
        scratch_types=[
            pltpu.VMEM((8, 16), jnp.int32),
            pltpu.VMEM((8, 16), jnp.int32),
            pltpu.VMEM((16, D_OUT), jnp.float32),
            pltpu.VMEM((16, D_OUT), jnp.float32),
            pltpu.SemaphoreType.DMA,
        ],
    )(_gather_body)
    return f(ys, pos)


def _combine_body(w_ref, y0_ref, y1_ref, out_ref):
    out_ref[...] = (w_ref[:, 0:1] * y0_ref[...] +
                    w_ref[:, 1:2] * y1_ref[...])


def _combine(wp, y0, y1):
    return pl.pallas_call(
        _combine_body,
        grid=(N // TT5,),
        in_specs=[
            pl.BlockSpec((TT5, TOP_K), lambda t: (t, 0)),
            pl.BlockSpec((TT5, D_OUT), lambda t: (t, 0)),
            pl.BlockSpec((TT5, D_OUT), lambda t: (t, 0)),
        ],
        out_specs=pl.BlockSpec((TT5, D_OUT), lambda t: (t, 0)),
        out_shape=jax.ShapeDtypeStruct((N, D_OUT), jnp.float32),
    )(wp, y0, y1)


@jax.jit
def kernel(x, Wr, br, W, b):
    wr_pad = jnp.zeros((D_IN, LANES), jnp.float32).at[:, :E].set(Wr)
    br_pad = jnp.zeros((1, LANES), jnp.float32).at[0, :E].set(br)
    idxp, wp = _router(x, wr_pad, br_pad)
    idx_flat = jnp.transpose(idxp).reshape(NPAIR)
    pos, eot, xs = _dispatch(idx_flat, x)
    ys = _ffn(xs, eot, W, b)
    y0, y1 = _combine_gather(ys, pos)
    return _combine(wp, y0, y1)


# speedup vs baseline: 1.0141x; 1.0141x over previous
import functools

import jax
import jax.numpy as jnp
from jax import lax
from jax.experimental import pallas as pl
from jax.experimental.pallas import tpu as pltpu
from jax.experimental.pallas import tpu_sc as plsc

N, D_IN, D_OUT, E, TOP_K = 4096, 1024, 1024, 8, 2
RT = 2048
LANES = 128
NEG = -1e30

NW = 32
NPAIR = TOP_K * N
CP = NPAIR // NW
TM = 256
P = NPAIR + E * TM
NT = P // TM
EOT_PAD = 48
TT5 = 2048


def _router_body(x_ref, wr_ref, br_ref, idx_ref, w_ref):
    lane = lax.broadcasted_iota(jnp.int32, (RT, LANES), 1)
    logits = jnp.dot(x_ref[...], wr_ref[...],
                     preferred_element_type=jnp.float32) + br_ref[...]
    logits = jnp.where(lane < E, logits, NEG)
    m1 = jnp.max(logits, axis=1, keepdims=True)
    i1 = jnp.min(jnp.where(logits == m1, lane, LANES), axis=1, keepdims=True)
    l2 = jnp.where(lane == i1, NEG, logits)
    m2 = jnp.max(l2, axis=1, keepdims=True)
    i2 = jnp.min(jnp.where(l2 == m2, lane, LANES), axis=1, keepdims=True)
    w0 = 1.0 / (1.0 + jnp.exp(m2 - m1))
    idx_ref[...] = jnp.concatenate([i1, i2], axis=1)
    w_ref[...] = jnp.concatenate([w0, 1.0 - w0], axis=1)


def _router(x, wr_pad, br_pad):
    return pl.pallas_call(
        _router_body,
        grid=(N // RT,),
        in_specs=[
            pl.BlockSpec((RT, D_IN), lambda t: (t, 0)),
            pl.BlockSpec((D_IN, LANES), lambda t: (0, 0)),
            pl.BlockSpec((1, LANES), lambda t: (0, 0)),
        ],
        out_specs=[
            pl.BlockSpec((RT, TOP_K), lambda t: (t, 0)),
            pl.BlockSpec((RT, TOP_K), lambda t: (t, 0)),
        ],
        out_shape=[
            jax.ShapeDtypeStruct((N, TOP_K), jnp.int32),
            jax.ShapeDtypeStruct((N, TOP_K), jnp.float32),
        ],
    )(x, wr_pad, br_pad)


def _dispatch_body(idx_hbm, x_hbm, pos_hbm, eot_hbm, xs_hbm,
                   idx_all, cur_v, off_v, ptot_v, pos_v, eot_v,
                   xbuf_a, xbuf_b, sem_ld, sem_st):
    wid = lax.axis_index("s") * 2 + lax.axis_index("c")
    lane16 = lax.broadcasted_iota(jnp.int32, (16,), 0)
    zeros = jnp.zeros((16,), jnp.int32)

    pltpu.sync_copy(idx_hbm, idx_all)

    def cbody(k, carry):
        hist, pre8 = carry
        v = idx_all[pl.ds(k * 16, 16)]
        ipre = jnp.where(k < wid * 16, 1, 0)
        hist = tuple(hist[e] + jnp.where(v == e, 1, 0) for e in range(E))
        pre8 = tuple(pre8[e] + jnp.where(v == e, ipre, 0) for e in range(E))
        return hist, pre8

    hist, pre8 = lax.fori_loop(
        0, NPAIR // 16, cbody,
        ((zeros,) * E, (zeros,) * E))
    tot = zeros
    pre = zeros
    for e in range(E):
        sel = lane16 == e
        tot = tot + jnp.where(sel, jnp.sum(hist[e]), 0)
        pre = pre + jnp.where(sel, jnp.sum(pre8[e]), 0)

    ptot = ((tot + (TM - 1)) >> 8) << 8
    off_incl = plsc.cumsum(ptot)
    off_excl = off_incl - ptot
    cur = off_excl + pre

    for k in range(CP // 16):
        v = idx_all[pl.ds((wid * 16 + k) * 16, 16)]
        cur_v[...] = cur
        base = plsc.load_gather(cur_v, [v])
        rank = zeros
        for e in range(E):
            m = v == e
            c01 = jnp.where(m, 1, 0)
            pref = plsc.cumsum(c01)
            rank = jnp.where(m, pref - 1, rank)
            cnt = jnp.sum(c01)
            cur = cur + jnp.where(lane16 == e, cnt, 0)
        pos_v[k] = base + rank

    pltpu.sync_copy(pos_v, pos_hbm.at[pl.ds(wid * 16, 16)])

    @pl.when(wid == 0)
    def _eot():
        off_v[...] = off_excl
        ptot_v[...] = ptot
        for k in range(EOT_PAD // 16):
            tstart = (lane16 + 16 * k) << 8
            eid = zeros
            for e in range(E):
                fe = jnp.full((16,), e, jnp.int32)
                oe = plsc.load_gather(off_v, [fe])
                pe = plsc.load_gather(ptot_v, [fe])
                inb = (tstart >= oe) & (tstart < oe + pe)
                eid = jnp.where(inb, e, eid)
            eot_v[pl.ds(16 * k, 16)] = eid
        pltpu.sync_copy(eot_v, eot_hbm)

    tok0 = wid * CP
    bufs = [xbuf_a, xbuf_b]
    nst = CP // 16
    lds = [None] * nst
    sts = [None] * nst
    lds[0] = pltpu.async_copy(x_hbm.at[pl.ds(tok0, 16)], bufs[0], sem_ld)
    for k in range(nst):
        lds[k].wait()
        if k + 1 < nst:
            if k >= 1:
                sts[k - 1].wait()
            lds[k + 1] = pltpu.async_copy(
                x_hbm.at[pl.ds(tok0 + (k + 1) * 16, 16)],
                bufs[(k + 1) % 2], sem_ld)
        sts[k] = pltpu.async_copy(bufs[k % 2], xs_hbm.at[pos_v.at[k]], sem_st)
    sts[nst - 2].wait()
    sts[nst - 1].wait()


def _dispatch(idx_flat, x):
    mesh = plsc.VectorSubcoreMesh(core_axis_name="c", subcore_axis_name="s")
    f = functools.partial(
        pl.kernel, mesh=mesh,
        compiler_params=pltpu.CompilerParams(needs_layout_passes=False),
        out_type=[
            jax.ShapeDtypeStruct((NPAIR // 16, 16), jnp.int32),
            jax.ShapeDtypeStruct((EOT_PAD,), jnp.int32),
            jax.ShapeDtypeStruct((P, D_IN), jnp.float32),
        ],
        scratch_types=[
            pltpu.VMEM((NPAIR,), jnp.int32),
            pltpu.VMEM((16,), jnp.int32),
            pltpu.VMEM((16,), jnp.int32),
            pltpu.VMEM((16,), jnp.int32),
            pltpu.VMEM((16, 16), jnp.int32),
            pltpu.VMEM((EOT_PAD,), jnp.int32),
            pltpu.VMEM((16, D_IN), jnp.float32),
            pltpu.VMEM((16, D_IN), jnp.float32),
            pltpu.SemaphoreType.DMA,
            pltpu.SemaphoreType.DMA,
        ],
    )(_dispatch_body)
    return f(idx_flat, x)


def _ffn_body(eot_ref, xs_ref, w_ref, b_ref, ys_ref):
    z = jnp.dot(xs_ref[...], w_ref[0],
                preferred_element_type=jnp.float32) + b_ref[0]
    ys_ref[...] = 0.5 * z * (1.0 + lax.erf(z * 0.7071067811865476))


def _ffn(xs, eot, W, b):
    return pl.pallas_call(
        _ffn_body,
        grid_spec=pltpu.PrefetchScalarGridSpec(
            num_scalar_prefetch=1,
            grid=(NT,),
            in_specs=[
                pl.BlockSpec((TM, D_IN), lambda t, eot: (t, 0)),
                pl.BlockSpec((1, D_IN, D_OUT), lambda t, eot: (eot[t], 0, 0)),
                pl.BlockSpec((1, 1, D_OUT), lambda t, eot: (eot[t], 0, 0)),
            ],
            out_specs=pl.BlockSpec((TM, D_OUT), lambda t, eot: (t, 0)),
        ),
        out_shape=jax.ShapeDtypeStruct((P, D_OUT), jnp.float32),
        compiler_params=pltpu.CompilerParams(
            dimension_semantics=("arbitrary",),
        ),
    )(eot, xs, W, b.reshape(E, 1, D_OUT))


def _gather_body(ys_hbm, pos_hbm, y0_hbm, y1_hbm,
                 p0_v, p1_v, ybuf_a, ybuf_b, sem):
    wid = lax.axis_index("s") * 2 + lax.axis_index("c")
    rows0 = N // 16
    pltpu.sync_copy(pos_hbm.at[pl.ds(wid * 8, 8)], p0_v)
    pltpu.sync_copy(pos_hbm.at[pl.ds(rows0 + wid * 8, 8)], p1_v)
    bufs = [ybuf_a, ybuf_b]
    nst = 16
    gs = [None] * nst

    def idx_row(j):
        return p0_v.at[j] if j < 8 else p1_v.at[j - 8]

    gs[0] = pltpu.async_copy(ys_hbm.at[idx_row(0)], bufs[0], sem)
    for j in range(nst):
        gs[j].wait()
        if j + 1 < nst:
            gs[j + 1] = pltpu.async_copy(
                ys_hbm.at[idx_row(j + 1)], bufs[(j + 1) % 2], sem)
        if j < 8:
            dst = y0_hbm.at[pl.ds(wid * 128 + j * 16, 16)]
        else:
            dst = y1_hbm.at[pl.ds(wid * 128 + (j - 8) * 16, 16)]
        pltpu.sync_copy(bufs[j % 2], dst)


def _combine_gather(ys, pos):
    mesh = plsc.VectorSubcoreMesh(core_axis_name="c", subcore_axis_name="s")
    f = functools.partial(
        pl.kernel, mesh=mesh,
        compiler_params=pltpu.CompilerParams(needs_layout_passes=False),
        out_type=[
            jax.ShapeDtypeStruct((N, D_OUT), jnp.float32),
            jax.ShapeDtypeStruct((N, D_OUT), jnp.float32),
        ],
        scratch_types=[
            pltpu.VMEM((8, 16), jnp.int32),
            pltpu.VMEM((8, 16), jnp.int32),
            pltpu.VMEM((16, D_OUT), jnp.float32),
            pltpu.VMEM((16, D_OUT), jnp.float32),
            pltpu.SemaphoreType.DMA,
        ],
    )(_gather_body)
    return f(ys, pos)


def _combine_body(w_ref, y0_ref, y1_ref, out_ref):
    out_ref[...] = (w_ref[:, 0:1] * y0_ref[...] +
                    w_ref[:, 1:2] * y1_ref[...])


def _combine(wp, y0, y1):
    return pl.pallas_call(
        _combine_body,
        grid=(N // TT5,),
        in_specs=[
            pl.BlockSpec((TT5, TOP_K), lambda t: (t, 0)),
            pl.BlockSpec((TT5, D_OUT), lambda t: (t, 0)),
            pl.BlockSpec((TT5, D_OUT), lambda t: (t, 0)),
        ],
        out_specs=pl.BlockSpec((TT5, D_OUT), lambda t: (t, 0)),
        out_shape=jax.ShapeDtypeStruct((N, D_OUT), jnp.float32),
    )(wp, y0, y1)


@jax.jit
def kernel(x, Wr, br, W, b):
    wr_pad = jnp.zeros((D_IN, LANES), jnp.float32).at[:, :E].set(Wr)
    br_pad = jnp.zeros((1, LANES), jnp.float32).at[0, :E].set(br)
    idxp, wp = _router(x, wr_pad, br_pad)
    idx_flat = jnp.transpose(idxp).reshape(NPAIR)
    pos, eot, xs = _dispatch(idx_flat, x)
    ys = _ffn(xs, eot, W, b)
    y0, y1 = _combine_gather(ys, pos)
    return _combine(wp, y0, y1)


# speedup vs baseline: 1.1597x; 1.1436x over previous
import functools

import jax
import jax.numpy as jnp
from jax import lax
from jax.experimental import pallas as pl
from jax.experimental.pallas import tpu as pltpu
from jax.experimental.pallas import tpu_sc as plsc

N, D_IN, D_OUT, E, TOP_K = 4096, 1024, 1024, 8, 2
RT = 2048
LANES = 128
NEG = -1e30

NW = 32
NPAIR = TOP_K * N
CP = NPAIR // NW
TM = 256
P = NPAIR + E * TM
NT = P // TM
EOT_PAD = 48
TT5 = 2048


def _router_body(x_ref, wr_ref, br_ref, idx_ref, w_ref):
    lane = lax.broadcasted_iota(jnp.int32, (RT, LANES), 1)
    logits = jnp.dot(x_ref[...], wr_ref[...],
                     preferred_element_type=jnp.float32) + br_ref[...]
    logits = jnp.where(lane < E, logits, NEG)
    m1 = jnp.max(logits, axis=1, keepdims=True)
    i1 = jnp.min(jnp.where(logits == m1, lane, LANES), axis=1, keepdims=True)
    l2 = jnp.where(lane == i1, NEG, logits)
    m2 = jnp.max(l2, axis=1, keepdims=True)
    i2 = jnp.min(jnp.where(l2 == m2, lane, LANES), axis=1, keepdims=True)
    w0 = 1.0 / (1.0 + jnp.exp(m2 - m1))
    idx_ref[...] = jnp.concatenate([i1, i2], axis=1)
    w_ref[...] = jnp.concatenate([w0, 1.0 - w0], axis=1)


def _router(x, wr_pad, br_pad):
    return pl.pallas_call(
        _router_body,
        grid=(N // RT,),
        in_specs=[
            pl.BlockSpec((RT, D_IN), lambda t: (t, 0)),
            pl.BlockSpec((D_IN, LANES), lambda t: (0, 0)),
            pl.BlockSpec((1, LANES), lambda t: (0, 0)),
        ],
        out_specs=[
            pl.BlockSpec((RT, TOP_K), lambda t: (t, 0)),
            pl.BlockSpec((RT, TOP_K), lambda t: (t, 0)),
        ],
        out_shape=[
            jax.ShapeDtypeStruct((N, TOP_K), jnp.int32),
            jax.ShapeDtypeStruct((N, TOP_K), jnp.float32),
        ],
    )(x, wr_pad, br_pad)


def _dispatch_body(idx_hbm, x_hbm, pos_hbm, eot_hbm, xs_hbm,
                   idx_all, cur_v, off_v, ptot_v, pos_v, eot_v,
                   xbuf_a, xbuf_b, sem_ld, sem_st):
    wid = lax.axis_index("s") * 2 + lax.axis_index("c")
    lane16 = lax.broadcasted_iota(jnp.int32, (16,), 0)
    zeros = jnp.zeros((16,), jnp.int32)

    pltpu.sync_copy(idx_hbm, idx_all)

    def cbody(k, carry):
        hist, pre8 = carry
        v = idx_all[pl.ds(k * 16, 16)]
        ipre = jnp.where(k < wid * 16, 1, 0)
        hist = tuple(hist[e] + jnp.where(v == e, 1, 0) for e in range(E))
        pre8 = tuple(pre8[e] + jnp.where(v == e, ipre, 0) for e in range(E))
        return hist, pre8

    hist, pre8 = lax.fori_loop(
        0, NPAIR // 16, cbody,
        ((zeros,) * E, (zeros,) * E))
    tot = zeros
    pre = zeros
    for e in range(E):
        sel = lane16 == e
        tot = tot + jnp.where(sel, jnp.sum(hist[e]), 0)
        pre = pre + jnp.where(sel, jnp.sum(pre8[e]), 0)

    ptot = ((tot + (TM - 1)) >> 8) << 8
    off_incl = plsc.cumsum(ptot)
    off_excl = off_incl - ptot
    cur = off_excl + pre

    for k in range(CP // 16):
        v = idx_all[pl.ds((wid * 16 + k) * 16, 16)]
        cur_v[...] = cur
        base = plsc.load_gather(cur_v, [v])
        rank = zeros
        for e in range(E):
            m = v == e
            c01 = jnp.where(m, 1, 0)
            pref = plsc.cumsum(c01)
            rank = jnp.where(m, pref - 1, rank)
            cnt = jnp.sum(c01)
            cur = cur + jnp.where(lane16 == e, cnt, 0)
        pos_v[k] = base + rank

    pltpu.sync_copy(pos_v, pos_hbm.at[pl.ds(wid * 16, 16)])

    @pl.when(wid == 0)
    def _eot():
        off_v[...] = off_excl
        ptot_v[...] = ptot
        for k in range(EOT_PAD // 16):
            tstart = (lane16 + 16 * k) << 8
            eid = zeros
            for e in range(E):
                fe = jnp.full((16,), e, jnp.int32)
                oe = plsc.load_gather(off_v, [fe])
                pe = plsc.load_gather(ptot_v, [fe])
                inb = (tstart >= oe) & (tstart < oe + pe)
                eid = jnp.where(inb, e, eid)
            eot_v[pl.ds(16 * k, 16)] = eid
        pltpu.sync_copy(eot_v, eot_hbm)

    tok0 = wid * CP
    bufs = [xbuf_a, xbuf_b]
    nst = CP // 16
    lds = [None] * nst
    sts = [None] * nst
    lds[0] = pltpu.async_copy(x_hbm.at[pl.ds(tok0, 16)], bufs[0], sem_ld)
    for k in range(nst):
        lds[k].wait()
        if k + 1 < nst:
            if k >= 1:
                sts[k - 1].wait()
            lds[k + 1] = pltpu.async_copy(
                x_hbm.at[pl.ds(tok0 + (k + 1) * 16, 16)],
                bufs[(k + 1) % 2], sem_ld)
        sts[k] = pltpu.async_copy(bufs[k % 2], xs_hbm.at[pos_v.at[k]], sem_st)
    sts[nst - 2].wait()
    sts[nst - 1].wait()


def _dispatch(idx_flat, x):
    mesh = plsc.VectorSubcoreMesh(core_axis_name="c", subcore_axis_name="s")
    f = functools.partial(
        pl.kernel, mesh=mesh,
        compiler_params=pltpu.CompilerParams(needs_layout_passes=False),
        out_type=[
            jax.ShapeDtypeStruct((NPAIR // 16, 16), jnp.int32),
            jax.ShapeDtypeStruct((EOT_PAD,), jnp.int32),
            jax.ShapeDtypeStruct((P, D_IN), jnp.float32),
        ],
        scratch_types=[
            pltpu.VMEM((NPAIR,), jnp.int32),
            pltpu.VMEM((16,), jnp.int32),
            pltpu.VMEM((16,), jnp.int32),
            pltpu.VMEM((16,), jnp.int32),
            pltpu.VMEM((16, 16), jnp.int32),
            pltpu.VMEM((EOT_PAD,), jnp.int32),
            pltpu.VMEM((16, D_IN), jnp.float32),
            pltpu.VMEM((16, D_IN), jnp.float32),
            pltpu.SemaphoreType.DMA,
            pltpu.SemaphoreType.DMA,
        ],
    )(_dispatch_body)
    return f(idx_flat, x)


def _ffn_body(eot_ref, xs_ref, w_ref, b_ref, ys_ref):
    z = jnp.dot(xs_ref[...], w_ref[0],
                preferred_element_type=jnp.float32) + b_ref[0]
    ys_ref[...] = 0.5 * z * (1.0 + lax.erf(z * 0.7071067811865476))


def _ffn(xs, eot, W, b):
    return pl.pallas_call(
        _ffn_body,
        grid_spec=pltpu.PrefetchScalarGridSpec(
            num_scalar_prefetch=1,
            grid=(NT,),
            in_specs=[
                pl.BlockSpec((TM, D_IN), lambda t, eot: (t, 0)),
                pl.BlockSpec((1, D_IN, D_OUT), lambda t, eot: (eot[t], 0, 0)),
                pl.BlockSpec((1, 1, D_OUT), lambda t, eot: (eot[t], 0, 0)),
            ],
            out_specs=pl.BlockSpec((TM, D_OUT), lambda t, eot: (t, 0)),
        ),
        out_shape=jax.ShapeDtypeStruct((P, D_OUT), jnp.float32),
        compiler_params=pltpu.CompilerParams(
            dimension_semantics=("arbitrary",),
        ),
    )(eot, xs, W, b.reshape(E, 1, D_OUT))


def _gather_body(ys_hbm, pos_hbm, w_hbm, out_hbm,
                 p0_v, p1_v, wv,
                 y0a, y0b, y1a, y1b, ob_a, ob_b, sem_g, sem_w):
    wid = lax.axis_index("s") * 2 + lax.axis_index("c")
    rows0 = N // 16
    tok0 = wid * 128
    pltpu.sync_copy(pos_hbm.at[pl.ds(wid * 8, 8)], p0_v)
    pltpu.sync_copy(pos_hbm.at[pl.ds(rows0 + wid * 8, 8)], p1_v)
    pltpu.sync_copy(w_hbm.at[pl.ds(tok0, 128)], wv)
    y0b_ = [y0a, y0b]
    y1b_ = [y1a, y1b]
    obuf = [ob_a, ob_b]
    zero16 = jnp.zeros((16,), jnp.int32)
    one16 = jnp.full((16,), 1, jnp.int32)

    nst = 8
    gs0 = [None] * nst
    gs1 = [None] * nst
    ws = [None] * nst
    gs0[0] = pltpu.async_copy(ys_hbm.at[p0_v.at[0]], y0b_[0], sem_g)
    gs1[0] = pltpu.async_copy(ys_hbm.at[p1_v.at[0]], y1b_[0], sem_g)
    for j in range(nst):
        gs0[j].wait()
        gs1[j].wait()
        if j + 1 < nst:
            gs0[j + 1] = pltpu.async_copy(
                ys_hbm.at[p0_v.at[j + 1]], y0b_[(j + 1) % 2], sem_g)
            gs1[j + 1] = pltpu.async_copy(
                ys_hbm.at[p1_v.at[j + 1]], y1b_[(j + 1) % 2], sem_g)
        if j >= 2:
            ws[j - 2].wait()
        y0r = y0b_[j % 2]
        y1r = y1b_[j % 2]
        our = obuf[j % 2]

        def tbody(tt, _, j=j, y0r=y0r, y1r=y1r, our=our):
            lrow = zero16 + (j * 16 + tt)
            w0s = plsc.load_gather(wv, [lrow, zero16])
            w1s = plsc.load_gather(wv, [lrow, one16])
            for s in range(D_OUT // 16):
                our[tt, pl.ds(s * 16, 16)] = (
                    w0s * y0r[tt, pl.ds(s * 16, 16)] +
                    w1s * y1r[tt, pl.ds(s * 16, 16)])
            return 0

        lax.fori_loop(0, 16, tbody, 0)
        ws[j] = pltpu.async_copy(
            our, out_hbm.at[pl.ds(tok0 + j * 16, 16)], sem_w)
    ws[nst - 2].wait()
    ws[nst - 1].wait()


def _combine_gather(ys, pos, wp):
    mesh = plsc.VectorSubcoreMesh(core_axis_name="c", subcore_axis_name="s")
    f = functools.partial(
        pl.kernel, mesh=mesh,
        compiler_params=pltpu.CompilerParams(needs_layout_passes=False),
        out_type=[
            jax.ShapeDtypeStruct((N, D_OUT), jnp.float32),
        ],
        scratch_types=[
            pltpu.VMEM((8, 16), jnp.int32),
            pltpu.VMEM((8, 16), jnp.int32),
            pltpu.VMEM((128, TOP_K), jnp.float32),
            pltpu.VMEM((16, D_OUT), jnp.float32),
            pltpu.VMEM((16, D_OUT), jnp.float32),
            pltpu.VMEM((16, D_OUT), jnp.float32),
            pltpu.VMEM((16, D_OUT), jnp.float32),
            pltpu.VMEM((16, D_OUT), jnp.float32),
            pltpu.VMEM((16, D_OUT), jnp.float32),
            pltpu.SemaphoreType.DMA,
            pltpu.SemaphoreType.DMA,
        ],
    )(_gather_body)
    (out,) = f(ys, pos, wp)
    return out


@jax.jit
def kernel(x, Wr, br, W, b):
    wr_pad = jnp.zeros((D_IN, LANES), jnp.float32).at[:, :E].set(Wr)
    br_pad = jnp.zeros((1, LANES), jnp.float32).at[0, :E].set(br)
    idxp, wp = _router(x, wr_pad, br_pad)
    idx_flat = jnp.transpose(idxp).reshape(NPAIR)
    pos, eot, xs = _dispatch(idx_flat, x)
    ys = _ffn(xs, eot, W, b)
    return _combine_gather(ys, pos, wp)


# speedup vs baseline: 1.2308x; 1.0613x over previous
import functools

import jax
import jax.numpy as jnp
from jax import lax
from jax.experimental import pallas as pl
from jax.experimental.pallas import tpu as pltpu
from jax.experimental.pallas import tpu_sc as plsc

N, D_IN, D_OUT, E, TOP_K = 4096, 1024, 1024, 8, 2
RT = 2048
LANES = 128
NEG = -1e30

NW = 32
NPAIR = TOP_K * N
CP = NPAIR // NW
TM = 256
P = NPAIR + E * TM
NT = P // TM
EOT_PAD = 48
TT5 = 2048


def _router_body(x_ref, wr_ref, br_ref, idx_ref, w_ref):
    lane = lax.broadcasted_iota(jnp.int32, (RT, LANES), 1)
    logits = jnp.dot(x_ref[...], wr_ref[...],
                     preferred_element_type=jnp.float32) + br_ref[...]
    logits = jnp.where(lane < E, logits, NEG)
    m1 = jnp.max(logits, axis=1, keepdims=True)
    i1 = jnp.min(jnp.where(logits == m1, lane, LANES), axis=1, keepdims=True)
    l2 = jnp.where(lane == i1, NEG, logits)
    m2 = jnp.max(l2, axis=1, keepdims=True)
    i2 = jnp.min(jnp.where(l2 == m2, lane, LANES), axis=1, keepdims=True)
    w0 = 1.0 / (1.0 + jnp.exp(m2 - m1))
    idx_ref[...] = jnp.concatenate([i1, i2], axis=1)
    w_ref[...] = jnp.concatenate([w0, 1.0 - w0], axis=1)


def _router(x, wr_pad, br_pad):
    return pl.pallas_call(
        _router_body,
        grid=(N // RT,),
        in_specs=[
            pl.BlockSpec((RT, D_IN), lambda t: (t, 0)),
            pl.BlockSpec((D_IN, LANES), lambda t: (0, 0)),
            pl.BlockSpec((1, LANES), lambda t: (0, 0)),
        ],
        out_specs=[
            pl.BlockSpec((RT, TOP_K), lambda t: (t, 0)),
            pl.BlockSpec((RT, TOP_K), lambda t: (t, 0)),
        ],
        out_shape=[
            jax.ShapeDtypeStruct((N, TOP_K), jnp.int32),
            jax.ShapeDtypeStruct((N, TOP_K), jnp.float32),
        ],
    )(x, wr_pad, br_pad)


def _dispatch_body(idx_hbm, x_hbm, pos_hbm, eot_hbm, xs_hbm,
                   idx_all, cur_v, off_v, ptot_v, pos_v, pos_v2, eot_v,
                   xbuf_a, xbuf_b, sem_ld, sem_st):
    wid = lax.axis_index("s") * 2 + lax.axis_index("c")
    lane16 = lax.broadcasted_iota(jnp.int32, (16,), 0)
    zeros = jnp.zeros((16,), jnp.int32)

    pltpu.sync_copy(idx_hbm, idx_all)

    tok0 = wid * CP
    bufs = [xbuf_a, xbuf_b]
    nst = CP // 32
    lds = [None] * nst
    sts = [None] * nst
    lds[0] = pltpu.async_copy(x_hbm.at[pl.ds(tok0, 32)], bufs[0], sem_ld)
    lds[1] = pltpu.async_copy(x_hbm.at[pl.ds(tok0 + 32, 32)], bufs[1],
                              sem_ld)

    def cbody(k, carry):
        hist, pre8 = carry
        v = idx_all[pl.ds(k * 16, 16)]
        ipre = jnp.where(k < wid * 16, 1, 0)
        hist = tuple(hist[e] + jnp.where(v == e, 1, 0) for e in range(E))
        pre8 = tuple(pre8[e] + jnp.where(v == e, ipre, 0) for e in range(E))
        return hist, pre8

    hist, pre8 = lax.fori_loop(
        0, NPAIR // 16, cbody,
        ((zeros,) * E, (zeros,) * E))
    tot = zeros
    pre = zeros
    for e in range(E):
        sel = lane16 == e
        tot = tot + jnp.where(sel, jnp.sum(hist[e]), 0)
        pre = pre + jnp.where(sel, jnp.sum(pre8[e]), 0)

    ptot = ((tot + (TM - 1)) >> 8) << 8
    off_incl = plsc.cumsum(ptot)
    off_excl = off_incl - ptot
    cur = off_excl + pre

    for k in range(CP // 16):
        v = idx_all[pl.ds((wid * 16 + k) * 16, 16)]
        cur_v[...] = cur
        base = plsc.load_gather(cur_v, [v])
        rank = zeros
        for e in range(E):
            m = v == e
            c01 = jnp.where(m, 1, 0)
            pref = plsc.cumsum(c01)
            rank = jnp.where(m, pref - 1, rank)
            cnt = jnp.sum(c01)
            cur = cur + jnp.where(lane16 == e, cnt, 0)
        dest = base + rank
        pos_v[k] = dest
        pos_v2[k // 2, pl.ds((k % 2) * 16, 16)] = dest

    pltpu.sync_copy(pos_v, pos_hbm.at[pl.ds(wid * 16, 16)])

    @pl.when(wid == 0)
    def _eot():
        off_v[...] = off_excl
        ptot_v[...] = ptot
        for k in range(EOT_PAD // 16):
            tstart = (lane16 + 16 * k) << 8
            eid = zeros
            for e in range(E):
                fe = jnp.full((16,), e, jnp.int32)
                oe = plsc.load_gather(off_v, [fe])
                pe = plsc.load_gather(ptot_v, [fe])
                inb = (tstart >= oe) & (tstart < oe + pe)
                eid = jnp.where(inb, e, eid)
            eot_v[pl.ds(16 * k, 16)] = eid
        pltpu.sync_copy(eot_v, eot_hbm)

    for k in range(nst):
        lds[k].wait()
        sts[k] = pltpu.async_copy(bufs[k % 2], xs_hbm.at[pos_v2.at[k]],
                                  sem_st)
        if k + 2 < nst:
            sts[k].wait()
            lds[k + 2] = pltpu.async_copy(
                x_hbm.at[pl.ds(tok0 + (k + 2) * 32, 32)],
                bufs[k % 2], sem_ld)
    sts[nst - 2].wait()
    sts[nst - 1].wait()


def _dispatch(idx_flat, x):
    mesh = plsc.VectorSubcoreMesh(core_axis_name="c", subcore_axis_name="s")
    f = functools.partial(
        pl.kernel, mesh=mesh,
        compiler_params=pltpu.CompilerParams(needs_layout_passes=False),
        out_type=[
            jax.ShapeDtypeStruct((NPAIR // 16, 16), jnp.int32),
            jax.ShapeDtypeStruct((EOT_PAD,), jnp.int32),
            jax.ShapeDtypeStruct((P, D_IN), jnp.float32),
        ],
        scratch_types=[
            pltpu.VMEM((NPAIR,), jnp.int32),
            pltpu.VMEM((16,), jnp.int32),
            pltpu.VMEM((16,), jnp.int32),
            pltpu.VMEM((16,), jnp.int32),
            pltpu.VMEM((16, 16), jnp.int32),
            pltpu.VMEM((8, 32), jnp.int32),
            pltpu.VMEM((EOT_PAD,), jnp.int32),
            pltpu.VMEM((32, D_IN), jnp.float32),
            pltpu.VMEM((32, D_IN), jnp.float32),
            pltpu.SemaphoreType.DMA,
            pltpu.SemaphoreType.DMA,
        ],
    )(_dispatch_body)
    return f(idx_flat, x)


def _ffn_body(eot_ref, xs_ref, w_ref, b_ref, ys_ref):
    z = jnp.dot(xs_ref[...], w_ref[0],
                preferred_element_type=jnp.float32) + b_ref[0]
    ys_ref[...] = 0.5 * z * (1.0 + lax.erf(z * 0.7071067811865476))


def _ffn(xs, eot, W, b):
    return pl.pallas_call(
        _ffn_body,
        grid_spec=pltpu.PrefetchScalarGridSpec(
            num_scalar_prefetch=1,
            grid=(NT,),
            in_specs=[
                pl.BlockSpec((TM, D_IN), lambda t, eot: (t, 0)),
                pl.BlockSpec((1, D_IN, D_OUT), lambda t, eot: (eot[t], 0, 0)),
                pl.BlockSpec((1, 1, D_OUT), lambda t, eot: (eot[t], 0, 0)),
            ],
            out_specs=pl.BlockSpec((TM, D_OUT), lambda t, eot: (t, 0)),
        ),
        out_shape=jax.ShapeDtypeStruct((P, D_OUT), jnp.float32),
        compiler_params=pltpu.CompilerParams(
            dimension_semantics=("arbitrary",),
        ),
    )(eot, xs, W, b.reshape(E, 1, D_OUT))


def _gather_body(ys_hbm, pos_hbm, w_hbm, out_hbm,
                 p0_v, p1_v, wv,
                 y0a, y0b, y1a, y1b, ob_a, ob_b, sem_g, sem_w):
    wid = lax.axis_index("s") * 2 + lax.axis_index("c")
    rows0 = N // 16
    tok0 = wid * 128
    pltpu.sync_copy(pos_hbm.at[pl.ds(wid * 8, 8)], p0_v)
    pltpu.sync_copy(pos_hbm.at[pl.ds(rows0 + wid * 8, 8)], p1_v)
    pltpu.sync_copy(w_hbm.at[pl.ds(tok0, 128)], wv)
    y0b_ = [y0a, y0b]
    y1b_ = [y1a, y1b]
    obuf = [ob_a, ob_b]
    zero16 = jnp.zeros((16,), jnp.int32)
    one16 = jnp.full((16,), 1, jnp.int32)

    nst = 8
    gs0 = [None] * nst
    gs1 = [None] * nst
    ws = [None] * nst
    gs0[0] = pltpu.async_copy(ys_hbm.at[p0_v.at[0]], y0b_[0], sem_g)
    gs1[0] = pltpu.async_copy(ys_hbm.at[p1_v.at[0]], y1b_[0], sem_g)
    for j in range(nst):
        gs0[j].wait()
        gs1[j].wait()
        if j + 1 < nst:
            gs0[j + 1] = pltpu.async_copy(
                ys_hbm.at[p0_v.at[j + 1]], y0b_[(j + 1) % 2], sem_g)
            gs1[j + 1] = pltpu.async_copy(
                ys_hbm.at[p1_v.at[j + 1]], y1b_[(j + 1) % 2], sem_g)
        if j >= 2:
            ws[j - 2].wait()
        y0r = y0b_[j % 2]
        y1r = y1b_[j % 2]
        our = obuf[j % 2]

        def tbody(tt, _, j=j, y0r=y0r, y1r=y1r, our=our):
            lrow = zero16 + (j * 16 + tt)
            w0s = plsc.load_gather(wv, [lrow, zero16])
            w1s = plsc.load_gather(wv, [lrow, one16])
            for s in range(D_OUT // 16):
                our[tt, pl.ds(s * 16, 16)] = (
                    w0s * y0r[tt, pl.ds(s * 16, 16)] +
                    w1s * y1r[tt, pl.ds(s * 16, 16)])
            return 0

        lax.fori_loop(0, 16, tbody, 0)
        ws[j] = pltpu.async_copy(
            our, out_hbm.at[pl.ds(tok0 + j * 16, 16)], sem_w)
    ws[nst - 2].wait()
    ws[nst - 1].wait()


def _combine_gather(ys, pos, wp):
    mesh = plsc.VectorSubcoreMesh(core_axis_name="c", subcore_axis_name="s")
    f = functools.partial(
        pl.kernel, mesh=mesh,
        compiler_params=pltpu.CompilerParams(needs_layout_passes=False),
        out_type=[
            jax.ShapeDtypeStruct((N, D_OUT), jnp.float32),
        ],
        scratch_types=[
            pltpu.VMEM((8, 16), jnp.int32),
            pltpu.VMEM((8, 16), jnp.int32),
            pltpu.VMEM((128, TOP_K), jnp.float32),
            pltpu.VMEM((16, D_OUT), jnp.float32),
            pltpu.VMEM((16, D_OUT), jnp.float32),
            pltpu.VMEM((16, D_OUT), jnp.float32),
            pltpu.VMEM((16, D_OUT), jnp.float32),
            pltpu.VMEM((16, D_OUT), jnp.float32),
            pltpu.VMEM((16, D_OUT), jnp.float32),
            pltpu.SemaphoreType.DMA,
            pltpu.SemaphoreType.DMA,
        ],
    )(_gather_body)
    (out,) = f(ys, pos, wp)
    return out


@jax.jit
def kernel(x, Wr, br, W, b):
    wr_pad = jnp.zeros((D_IN, LANES), jnp.float32).at[:, :E].set(Wr)
    br_pad = jnp.zeros((1, LANES), jnp.float32).at[0, :E].set(br)
    idxp, wp = _router(x, wr_pad, br_pad)
    idx_flat = jnp.transpose(idxp).reshape(NPAIR)
    pos, eot, xs = _dispatch(idx_flat, x)
    ys = _ffn(xs, eot, W, b)
    return _combine_gather(ys, pos, wp)
